# baseline (speedup 1.0000x reference)
import functools

import jax
import jax.numpy as jnp
from jax import lax
from jax.experimental import pallas as pl
from jax.experimental.pallas import tpu as pltpu, tpu_sc as plsc

_N = 10000
_NP = 10240
_E = 320000
_B = 4096
_NC = 2
_NS = 16
_NW = _NC * _NS

_HIGH = jax.lax.Precision.HIGHEST


def _dot(a, b):
    return jnp.dot(a, b, precision=_HIGH, preferred_element_type=jnp.float32)


def _make_segsum(W, CH=80, NB=4):
    EPW = _E // _NW
    NCH = EPW // CH
    RPT = _NP // _NS
    mesh = plsc.VectorSubcoreMesh(core_axis_name="c", subcore_axis_name="s")

    NIT = NCH // (2 * NB)
    TL0 = NIT * 2 * NB

    @functools.partial(
        pl.kernel,
        out_type=jax.ShapeDtypeStruct((_NC, _NP, W), jnp.float32),
        mesh=mesh,
        scratch_types=[
            [pltpu.VMEM((CH,), jnp.int32)] * (2 * NB),
            [pltpu.VMEM((CH,), jnp.int32)] * (2 * NB),
            [pltpu.VMEM((CH, W), jnp.float32)] * NB,
            pltpu.VMEM_SHARED((_NP, W), jnp.float32),
            [pltpu.SemaphoreType.DMA] * (2 * NB),
            [pltpu.SemaphoreType.DMA] * (2 * NB),
            [pltpu.SemaphoreType.DMA] * NB,
            [pltpu.SemaphoreType.DMA] * (2 * NB),
        ],
    )
    def seg(h_hbm, src_hbm, dst_hbm, zeros_hbm, out_hbm,
            sb, db, rb, accum, isem, jsem, gsem, ssem):
        cid = lax.axis_index("c")
        sid = lax.axis_index("s")
        wid = sid * _NC + cid
        base = wid * EPW
        pltpu.sync_copy(zeros_hbm, accum.at[pl.ds(sid * RPT, RPT)])
        plsc.subcore_barrier()

        def fetch_idx(k, off):
            off = pl.multiple_of(off, 8)
            pltpu.async_copy(src_hbm.at[pl.ds(off, CH)], sb[k], isem[k])
            pltpu.async_copy(dst_hbm.at[pl.ds(off, CH)], db[k], jsem[k])

        def wait_idx(k):
            pltpu.make_async_copy(src_hbm.at[pl.ds(0, CH)], sb[k],
                                  isem[k]).wait()

        def drain_scat(k, b):
            pltpu.make_async_copy(rb[b], accum.at[db[k]], ssem[k]).wait()

        for b in range(NB):
            fetch_idx(b, base + b * CH)

        def group(g, carry):
            c0 = base + g * (2 * NB * CH)
            for s in range(2):
                ns = 1 - s
                gdescs = []
                for b in range(NB):
                    kn = ns * NB + b
                    if s == 0:
                        @pl.when(g > 0)
                        def _drain():
                            drain_scat(kn, b)
                    else:
                        drain_scat(kn, b)
                    fetch_idx(kn, c0 + (s + 1) * NB * CH + b * CH)
                for b in range(NB):
                    k = s * NB + b
                    wait_idx(k)
                    gdescs.append(pltpu.async_copy(h_hbm.at[sb[k]], rb[b],
                                                   gsem[b]))
                for b in range(NB):
                    k = s * NB + b
                    gdescs[b].wait()
                    pltpu.make_async_copy(dst_hbm.at[pl.ds(0, CH)], db[k],
                                          jsem[k]).wait()
                    pltpu.async_copy(rb[b], accum.at[db[k]], ssem[k],
                                     add=True)
            return carry

        lax.fori_loop(0, NIT, group, 0)
        for b in range(NB):
            drain_scat(NB + b, b)
        for b in range(NB):
            wait_idx(b)
            pltpu.async_copy(h_hbm.at[sb[b]], rb[b], gsem[b]).wait()
            pltpu.make_async_copy(dst_hbm.at[pl.ds(0, CH)], db[b],
                                  jsem[b]).wait()
            pltpu.sync_copy(rb[b], accum.at[db[b]], add=True)
        for t in range(NCH - TL0 - NB):
            off = pl.multiple_of(base + (TL0 + NB + t) * CH, 8)
            pltpu.sync_copy(src_hbm.at[pl.ds(off, CH)], sb[t])
            pltpu.sync_copy(dst_hbm.at[pl.ds(off, CH)], db[t])
            pltpu.async_copy(h_hbm.at[sb[t]], rb[t], gsem[t]).wait()
            pltpu.sync_copy(rb[t], accum.at[db[t]], add=True)
        plsc.subcore_barrier()
        pltpu.sync_copy(accum.at[pl.ds(sid * RPT, RPT)],
                        out_hbm.at[cid, pl.ds(sid * RPT, RPT)])

    return seg


def _make_pair_gather(W):
    BPW = _B // _NW
    mesh = plsc.VectorSubcoreMesh(core_axis_name="c", subcore_axis_name="s")

    @functools.partial(
        pl.kernel,
        out_type=(jax.ShapeDtypeStruct((_B, W), jnp.float32),
                  jax.ShapeDtypeStruct((_B, W), jnp.float32)),
        mesh=mesh,
        scratch_types=[
            pltpu.VMEM((BPW,), jnp.int32),
            pltpu.VMEM((BPW, W), jnp.float32),
            pltpu.SemaphoreType.DMA,
        ],
    )
    def gat(tab_hbm, bid_hbm, aid_hbm, outb_hbm, outa_hbm, idx, rows, sem):
        cid = lax.axis_index("c")
        sid = lax.axis_index("s")
        base = (sid * _NC + cid) * BPW
        pltpu.sync_copy(bid_hbm.at[pl.ds(base, BPW)], idx)
        pltpu.async_copy(tab_hbm.at[idx], rows, sem).wait()
        pltpu.sync_copy(rows, outb_hbm.at[pl.ds(base, BPW)])
        pltpu.sync_copy(aid_hbm.at[pl.ds(base, BPW)], idx)
        pltpu.async_copy(tab_hbm.at[idx], rows, sem).wait()
        pltpu.sync_copy(rows, outa_hbm.at[pl.ds(base, BPW)])

    return gat


def _tc1(part1, x, w1l, b1, w1r, w2l):
    RB = 1000

    def body(p_ref, x_ref, wl_ref, b_ref, wr_ref, w2_ref, z1_ref, h2_ref):
        agg = p_ref[0] + p_ref[1]
        z1 = jnp.maximum(
            _dot(agg, wl_ref[...]) + b_ref[...] + _dot(x_ref[...], wr_ref[...]),
            0.0)
        z1_ref[...] = z1
        h2_ref[...] = _dot(z1, w2_ref[...])

    return pl.pallas_call(
        body,
        grid=(_N // RB,),
        in_specs=[
            pl.BlockSpec((2, RB, 128), lambda i: (0, i, 0)),
            pl.BlockSpec((RB, 128), lambda i: (i, 0)),
            pl.BlockSpec((128, 256), lambda i: (0, 0)),
            pl.BlockSpec((1, 256), lambda i: (0, 0)),
            pl.BlockSpec((128, 256), lambda i: (0, 0)),
            pl.BlockSpec((256, 128), lambda i: (0, 0)),
        ],
        out_specs=[pl.BlockSpec((RB, 256), lambda i: (i, 0)),
                   pl.BlockSpec((RB, 128), lambda i: (i, 0))],
        out_shape=[jax.ShapeDtypeStruct((_N, 256), jnp.float32),
                   jax.ShapeDtypeStruct((_N, 128), jnp.float32)],
    )(part1, x, w1l, b1, w1r, w2l)


def _tc2(part2, z1, w2r, b2):
    RB = 1000

    def body(p_ref, z1_ref, wr_ref, b_ref, z2_ref):
        z2_ref[...] = jnp.maximum(
            p_ref[0] + p_ref[1] + b_ref[...] + _dot(z1_ref[...], wr_ref[...]),
            0.0)

    return pl.pallas_call(
        body,
        grid=(_N // RB,),
        in_specs=[
            pl.BlockSpec((2, RB, 128), lambda i: (0, i, 0)),
            pl.BlockSpec((RB, 256), lambda i: (i, 0)),
            pl.BlockSpec((256, 128), lambda i: (0, 0)),
            pl.BlockSpec((1, 128), lambda i: (0, 0)),
        ],
        out_specs=pl.BlockSpec((RB, 128), lambda i: (i, 0)),
        out_shape=jax.ShapeDtypeStruct((_N, 128), jnp.float32),
    )(part2, z1, w2r, b2)


def _tc3(part3, z2, w3l_p, w3r_p, b3_p):
    NPG = 10016
    RB = 2504

    def body(p_ref, z2_ref, wl_ref, wr_ref, b_ref, tab_ref):
        i = pl.program_id(0)
        row = i * RB + lax.broadcasted_iota(jnp.int32, (RB, 1), 0)
        agg = p_ref[0] + p_ref[1]
        val = jnp.tanh(
            _dot(agg, wl_ref[...]) + b_ref[...]
            + _dot(z2_ref[...], wr_ref[...]))
        tab_ref[...] = jnp.where(row < _N, val, 0.0)

    return pl.pallas_call(
        body,
        grid=(NPG // RB,),
        in_specs=[
            pl.BlockSpec((2, RB, 128), lambda i: (0, i, 0)),
            pl.BlockSpec((RB, 128), lambda i: (i, 0)),
            pl.BlockSpec((128, 128), lambda i: (0, 0)),
            pl.BlockSpec((128, 128), lambda i: (0, 0)),
            pl.BlockSpec((1, 128), lambda i: (0, 0)),
        ],
        out_specs=pl.BlockSpec((RB, 128), lambda i: (i, 0)),
        out_shape=jax.ShapeDtypeStruct((NPG, 128), jnp.float32),
    )(part3, z2, w3l_p, w3r_p, b3_p)


def _tc_vae(act_vec, zb, za, eps_a, eps_g, ws):
    def body(act_ref, zb_ref, za_ref, ea_ref, eg_ref,
             a_w1, a_b1, a_w2, a_b2, amu_w, amu_b, alv_w, alv_b,
             go_w1a, go_w1b, go_b1, go_w2, go_b2,
             gmu_w, gmu_b, glv_w, glv_b,
             d_w1a, d_w1b, d_b1, d_w2, d_b2, d_w3, d_b3, d_w4, d_b4,
             recon_ref, akl_ref, gkl_ref):
        act = act_ref[...]
        h = _dot(jnp.maximum(_dot(act, a_w1[...]) + a_b1[...], 0.0),
                 a_w2[...]) + a_b2[...]
        hr = jnp.maximum(h, 0.0)
        a_mu = _dot(hr, amu_w[...]) + amu_b[...]
        a_lv = _dot(hr, alv_w[...]) + alv_b[...]
        act_z = ea_ref[...] * jnp.exp(0.5 * a_lv) + a_mu

        gv1 = jnp.maximum(
            _dot(act_z, go_w1a[...]) + _dot(zb_ref[...][:, :64], go_w1b[...])
            + go_b1[...], 0.0)
        gv = _dot(gv1, go_w2[...]) + go_b2[...]
        gvr = jnp.maximum(gv, 0.0)
        g_mu = _dot(gvr, gmu_w[...]) + gmu_b[...]
        g_lv = _dot(gvr, glv_w[...]) + glv_b[...]
        graph_z = eg_ref[...] * jnp.exp(0.5 * g_lv) + g_mu

        t1 = jnp.maximum(
            _dot(act_z, d_w1a[...]) + _dot(graph_z, d_w1b[...]) + d_b1[...],
            0.0)
        t2 = _dot(t1, d_w2[...]) + d_b2[...]
        t3 = jnp.maximum(t2, 0.0)
        t4 = jnp.maximum(_dot(t3, d_w3[...]) + d_b3[...], 0.0)
        dec = _dot(t4, d_w4[...]) + d_b4[...]

        diff = dec - za_ref[...][:, :64]
        recon_ref[...] = (jnp.sum(diff * diff)
                          * (1.0 / diff.size)).reshape(1, 1)
        akl_ref[...] = (-0.5 * jnp.sum(1.0 + a_lv - a_mu * a_mu
                                       - jnp.exp(a_lv))).reshape(1, 1)
        gkl_ref[...] = (-0.5 * jnp.sum(1.0 + g_lv - g_mu * g_mu
                                       - jnp.exp(g_lv))).reshape(1, 1)

    outs = pl.pallas_call(
        body,
        out_shape=[jax.ShapeDtypeStruct((1, 1), jnp.float32)] * 3,
    )(act_vec, zb, za, eps_a, eps_g, *ws)
    return outs


_segsum128 = _make_segsum(128)
_pair_gather = _make_pair_gather(128)


def kernel(act_vec, x, params, before_id, after_id, edge_index):
    p = params
    src = edge_index[0]
    dst = edge_index[1]
    zeros128 = jnp.zeros((_NP // _NS, 128), jnp.float32)

    def b(v):
        return v.reshape(1, -1)

    def pad64(w):
        return jnp.pad(w, [(0, 0)] * (w.ndim - 1) + [(0, 64)])

    part1 = _segsum128(x, src, dst, zeros128)
    z1, h2 = _tc1(part1, x, p['g1_wl'], b(p['g1_bl']), p['g1_wr'], p['g2_wl'])
    part2 = _segsum128(h2, src, dst, zeros128)
    z2 = _tc2(part2, z1, p['g2_wr'], b(p['g2_bl']))
    part3 = _segsum128(z2, src, dst, zeros128)
    tab = _tc3(part3, z2, pad64(p['g3_wl']), pad64(p['g3_wr']),
               pad64(b(p['g3_bl'])))

    zb, za = _pair_gather(tab, before_id, after_id)

    eps_a = jax.random.normal(jax.random.key(42), (_B, 32), jnp.float32)
    eps_g = jax.random.normal(jax.random.key(43), (_B, 64), jnp.float32)
    ws = [
        p['a_w1'], b(p['a_b1']), p['a_w2'], b(p['a_b2']),
        p['amu_w'], b(p['amu_b']), p['alv_w'], b(p['alv_b']),
        p['go_w1'][:32], p['go_w1'][32:], b(p['go_b1']),
        p['go_w2'], b(p['go_b2']),
        p['gmu_w'], b(p['gmu_b']), p['glv_w'], b(p['glv_b']),
        p['d_w1'][:32], p['d_w1'][32:], b(p['d_b1']),
        p['d_w2'], b(p['d_b2']), p['d_w3'], b(p['d_b3']),
        p['d_w4'], b(p['d_b4']),
    ]
    recon, akl, gkl = _tc_vae(act_vec, zb, za, eps_a, eps_g, ws)
    return (recon[0, 0], akl[0, 0], gkl[0, 0])


# speedup vs baseline: 1.0567x; 1.0567x over previous
import functools

import jax
import jax.numpy as jnp
from jax import lax
from jax.experimental import pallas as pl
from jax.experimental.pallas import tpu as pltpu, tpu_sc as plsc

_N = 10000
_NP = 10240
_E = 320000
_B = 4096
_NC = 2
_NS = 16
_NW = _NC * _NS

_HIGH = jax.lax.Precision.HIGHEST


def _dot(a, b):
    return jnp.dot(a, b, precision=_HIGH, preferred_element_type=jnp.float32)


def _make_segsum(W, CH=80, NB=4):
    EPW = _E // _NW
    NCH = EPW // CH
    RPT = _NP // _NS
    mesh = plsc.VectorSubcoreMesh(core_axis_name="c", subcore_axis_name="s")

    NG = NCH // NB
    TAIL = NCH - NG * NB

    @functools.partial(
        pl.kernel,
        out_type=jax.ShapeDtypeStruct((_NC, _NP, W), jnp.float32),
        mesh=mesh,
        scratch_types=[
            [pltpu.VMEM((CH,), jnp.int32)] * NB,
            [pltpu.VMEM((CH,), jnp.int32)] * NB,
            [pltpu.VMEM((CH, W), jnp.float32)] * NB,
            pltpu.VMEM_SHARED((_NP, W), jnp.float32),
            [pltpu.SemaphoreType.DMA] * NB,
            [pltpu.SemaphoreType.DMA] * NB,
            [pltpu.SemaphoreType.DMA] * NB,
            [pltpu.SemaphoreType.DMA] * NB,
        ],
    )
    def seg(h_hbm, src_hbm, dst_hbm, zeros_hbm, out_hbm,
            sb, db, rb, accum, isem, jsem, gsem, ssem):
        cid = lax.axis_index("c")
        sid = lax.axis_index("s")
        wid = sid * _NC + cid
        base = wid * EPW
        pltpu.sync_copy(zeros_hbm, accum.at[pl.ds(sid * RPT, RPT)])
        plsc.subcore_barrier()

        def group(g, carry):
            c0 = base + g * (NB * CH)
            idescs, jdescs, gdescs = [], [], []
            for b in range(NB):
                @pl.when(g > 0)
                def _drain():
                    pltpu.make_async_copy(rb[b], accum.at[db[b]],
                                          ssem[b]).wait()

                off = pl.multiple_of(c0 + b * CH, 8)
                idescs.append(pltpu.async_copy(
                    src_hbm.at[pl.ds(off, CH)], sb[b], isem[b]))
                jdescs.append(pltpu.async_copy(
                    dst_hbm.at[pl.ds(off, CH)], db[b], jsem[b]))
            for b in range(NB):
                idescs[b].wait()
                gdescs.append(pltpu.async_copy(h_hbm.at[sb[b]], rb[b],
                                               gsem[b]))
            for b in range(NB):
                gdescs[b].wait()
                jdescs[b].wait()
                pltpu.async_copy(rb[b], accum.at[db[b]], ssem[b], add=True)
            return carry

        lax.fori_loop(0, NG, group, 0)
        for b in range(NB):
            pltpu.make_async_copy(rb[b], accum.at[db[b]], ssem[b]).wait()
        for t in range(TAIL):
            off = pl.multiple_of(base + (NG * NB + t) * CH, 8)
            pltpu.sync_copy(src_hbm.at[pl.ds(off, CH)], sb[t])
            pltpu.sync_copy(dst_hbm.at[pl.ds(off, CH)], db[t])
            pltpu.async_copy(h_hbm.at[sb[t]], rb[t], gsem[t]).wait()
            pltpu.sync_copy(rb[t], accum.at[db[t]], add=True)
        plsc.subcore_barrier()
        pltpu.sync_copy(accum.at[pl.ds(sid * RPT, RPT)],
                        out_hbm.at[cid, pl.ds(sid * RPT, RPT)])

    return seg


def _make_pair_gather(W):
    BPW = _B // _NW
    mesh = plsc.VectorSubcoreMesh(core_axis_name="c", subcore_axis_name="s")

    @functools.partial(
        pl.kernel,
        out_type=(jax.ShapeDtypeStruct((_B, W), jnp.float32),
                  jax.ShapeDtypeStruct((_B, W), jnp.float32)),
        mesh=mesh,
        scratch_types=[
            pltpu.VMEM((BPW,), jnp.int32),
            pltpu.VMEM((BPW, W), jnp.float32),
            pltpu.SemaphoreType.DMA,
        ],
    )
    def gat(tab_hbm, bid_hbm, aid_hbm, outb_hbm, outa_hbm, idx, rows, sem):
        cid = lax.axis_index("c")
        sid = lax.axis_index("s")
        base = (sid * _NC + cid) * BPW
        pltpu.sync_copy(bid_hbm.at[pl.ds(base, BPW)], idx)
        pltpu.async_copy(tab_hbm.at[idx], rows, sem).wait()
        pltpu.sync_copy(rows, outb_hbm.at[pl.ds(base, BPW)])
        pltpu.sync_copy(aid_hbm.at[pl.ds(base, BPW)], idx)
        pltpu.async_copy(tab_hbm.at[idx], rows, sem).wait()
        pltpu.sync_copy(rows, outa_hbm.at[pl.ds(base, BPW)])

    return gat


def _tc1(part1, x, w1l, b1, w1r, w2l):
    RB = 1000

    def body(p_ref, x_ref, wl_ref, b_ref, wr_ref, w2_ref, z1_ref, h2_ref):
        agg = p_ref[0] + p_ref[1]
        z1 = jnp.maximum(
            _dot(agg, wl_ref[...]) + b_ref[...] + _dot(x_ref[...], wr_ref[...]),
            0.0)
        z1_ref[...] = z1
        h2_ref[...] = _dot(z1, w2_ref[...])

    return pl.pallas_call(
        body,
        grid=(_N // RB,),
        in_specs=[
            pl.BlockSpec((2, RB, 128), lambda i: (0, i, 0)),
            pl.BlockSpec((RB, 128), lambda i: (i, 0)),
            pl.BlockSpec((128, 256), lambda i: (0, 0)),
            pl.BlockSpec((1, 256), lambda i: (0, 0)),
            pl.BlockSpec((128, 256), lambda i: (0, 0)),
            pl.BlockSpec((256, 128), lambda i: (0, 0)),
        ],
        out_specs=[pl.BlockSpec((RB, 256), lambda i: (i, 0)),
                   pl.BlockSpec((RB, 128), lambda i: (i, 0))],
        out_shape=[jax.ShapeDtypeStruct((_N, 256), jnp.float32),
                   jax.ShapeDtypeStruct((_N, 128), jnp.float32)],
    )(part1, x, w1l, b1, w1r, w2l)


def _tc2(part2, z1, w2r, b2):
    RB = 1000

    def body(p_ref, z1_ref, wr_ref, b_ref, z2_ref):
        z2_ref[...] = jnp.maximum(
            p_ref[0] + p_ref[1] + b_ref[...] + _dot(z1_ref[...], wr_ref[...]),
            0.0)

    return pl.pallas_call(
        body,
        grid=(_N // RB,),
        in_specs=[
            pl.BlockSpec((2, RB, 128), lambda i: (0, i, 0)),
            pl.BlockSpec((RB, 256), lambda i: (i, 0)),
            pl.BlockSpec((256, 128), lambda i: (0, 0)),
            pl.BlockSpec((1, 128), lambda i: (0, 0)),
        ],
        out_specs=pl.BlockSpec((RB, 128), lambda i: (i, 0)),
        out_shape=jax.ShapeDtypeStruct((_N, 128), jnp.float32),
    )(part2, z1, w2r, b2)


def _tc3(part3, z2, w3l_p, w3r_p, b3_p):
    NPG = 10016
    RB = 2504

    def body(p_ref, z2_ref, wl_ref, wr_ref, b_ref, tab_ref):
        i = pl.program_id(0)
        row = i * RB + lax.broadcasted_iota(jnp.int32, (RB, 1), 0)
        agg = p_ref[0] + p_ref[1]
        val = jnp.tanh(
            _dot(agg, wl_ref[...]) + b_ref[...]
            + _dot(z2_ref[...], wr_ref[...]))
        tab_ref[...] = jnp.where(row < _N, val, 0.0)

    return pl.pallas_call(
        body,
        grid=(NPG // RB,),
        in_specs=[
            pl.BlockSpec((2, RB, 128), lambda i: (0, i, 0)),
            pl.BlockSpec((RB, 128), lambda i: (i, 0)),
            pl.BlockSpec((128, 128), lambda i: (0, 0)),
            pl.BlockSpec((128, 128), lambda i: (0, 0)),
            pl.BlockSpec((1, 128), lambda i: (0, 0)),
        ],
        out_specs=pl.BlockSpec((RB, 128), lambda i: (i, 0)),
        out_shape=jax.ShapeDtypeStruct((NPG, 128), jnp.float32),
    )(part3, z2, w3l_p, w3r_p, b3_p)


def _tc_vae(act_vec, zb, za, eps_a, eps_g, ws):
    def body(act_ref, zb_ref, za_ref, ea_ref, eg_ref,
             a_w1, a_b1, a_w2, a_b2, amu_w, amu_b, alv_w, alv_b,
             go_w1a, go_w1b, go_b1, go_w2, go_b2,
             gmu_w, gmu_b, glv_w, glv_b,
             d_w1a, d_w1b, d_b1, d_w2, d_b2, d_w3, d_b3, d_w4, d_b4,
             recon_ref, akl_ref, gkl_ref):
        act = act_ref[...]
        h = _dot(jnp.maximum(_dot(act, a_w1[...]) + a_b1[...], 0.0),
                 a_w2[...]) + a_b2[...]
        hr = jnp.maximum(h, 0.0)
        a_mu = _dot(hr, amu_w[...]) + amu_b[...]
        a_lv = _dot(hr, alv_w[...]) + alv_b[...]
        act_z = ea_ref[...] * jnp.exp(0.5 * a_lv) + a_mu

        gv1 = jnp.maximum(
            _dot(act_z, go_w1a[...]) + _dot(zb_ref[...][:, :64], go_w1b[...])
            + go_b1[...], 0.0)
        gv = _dot(gv1, go_w2[...]) + go_b2[...]
        gvr = jnp.maximum(gv, 0.0)
        g_mu = _dot(gvr, gmu_w[...]) + gmu_b[...]
        g_lv = _dot(gvr, glv_w[...]) + glv_b[...]
        graph_z = eg_ref[...] * jnp.exp(0.5 * g_lv) + g_mu

        t1 = jnp.maximum(
            _dot(act_z, d_w1a[...]) + _dot(graph_z, d_w1b[...]) + d_b1[...],
            0.0)
        t2 = _dot(t1, d_w2[...]) + d_b2[...]
        t3 = jnp.maximum(t2, 0.0)
        t4 = jnp.maximum(_dot(t3, d_w3[...]) + d_b3[...], 0.0)
        dec = _dot(t4, d_w4[...]) + d_b4[...]

        diff = dec - za_ref[...][:, :64]
        recon_ref[...] = (jnp.sum(diff * diff)
                          * (1.0 / diff.size)).reshape(1, 1)
        akl_ref[...] = (-0.5 * jnp.sum(1.0 + a_lv - a_mu * a_mu
                                       - jnp.exp(a_lv))).reshape(1, 1)
        gkl_ref[...] = (-0.5 * jnp.sum(1.0 + g_lv - g_mu * g_mu
                                       - jnp.exp(g_lv))).reshape(1, 1)

    outs = pl.pallas_call(
        body,
        out_shape=[jax.ShapeDtypeStruct((1, 1), jnp.float32)] * 3,
    )(act_vec, zb, za, eps_a, eps_g, *ws)
    return outs


_segsum128 = _make_segsum(128)
_pair_gather = _make_pair_gather(128)


def kernel(act_vec, x, params, before_id, after_id, edge_index):
    p = params
    src = edge_index[0]
    dst = edge_index[1]
    zeros128 = jnp.zeros((_NP // _NS, 128), jnp.float32)

    def b(v):
        return v.reshape(1, -1)

    def pad64(w):
        return jnp.pad(w, [(0, 0)] * (w.ndim - 1) + [(0, 64)])

    part1 = _segsum128(x, src, dst, zeros128)
    z1, h2 = _tc1(part1, x, p['g1_wl'], b(p['g1_bl']), p['g1_wr'], p['g2_wl'])
    part2 = _segsum128(h2, src, dst, zeros128)
    z2 = _tc2(part2, z1, p['g2_wr'], b(p['g2_bl']))
    part3 = _segsum128(z2, src, dst, zeros128)
    tab = _tc3(part3, z2, pad64(p['g3_wl']), pad64(p['g3_wr']),
               pad64(b(p['g3_bl'])))

    zb, za = _pair_gather(tab, before_id, after_id)

    eps_a = jax.random.normal(jax.random.key(42), (_B, 32), jnp.float32)
    eps_g = jax.random.normal(jax.random.key(43), (_B, 64), jnp.float32)
    ws = [
        p['a_w1'], b(p['a_b1']), p['a_w2'], b(p['a_b2']),
        p['amu_w'], b(p['amu_b']), p['alv_w'], b(p['alv_b']),
        p['go_w1'][:32], p['go_w1'][32:], b(p['go_b1']),
        p['go_w2'], b(p['go_b2']),
        p['gmu_w'], b(p['gmu_b']), p['glv_w'], b(p['glv_b']),
        p['d_w1'][:32], p['d_w1'][32:], b(p['d_b1']),
        p['d_w2'], b(p['d_b2']), p['d_w3'], b(p['d_b3']),
        p['d_w4'], b(p['d_b4']),
    ]
    recon, akl, gkl = _tc_vae(act_vec, zb, za, eps_a, eps_g, ws)
    return (recon[0, 0], akl[0, 0], gkl[0, 0])
